# baseline (speedup 1.0000x reference)
import functools

import jax
import jax.numpy as jnp
from jax import lax
from jax.experimental import pallas as pl
from jax.experimental.pallas import tpu as pltpu
from jax.experimental.pallas import tpu_sc as plsc

N = 10000
E = 320000
D = 128
G = 16
NC, NS = 2, 16
NW = NC * NS
CH = 128
NCH = 80
E_W = NCH * CH
E_PAD = NW * E_W
NP = N + 16
NPD = N + 640

_mesh = plsc.VectorSubcoreMesh(core_axis_name="c", subcore_axis_name="s")


@functools.partial(
    pl.kernel,
    mesh=_mesh,
    out_type=jax.ShapeDtypeStruct((NC * N,), jnp.float32),
    scratch_types=[
        pltpu.VMEM((NCH, CH), jnp.int32),
        pltpu.VMEM((CH,), jnp.float32),
        pltpu.VMEM((N,), jnp.float32),
        pltpu.VMEM_SHARED((NPD,), jnp.float32),
        pltpu.SemaphoreType.DMA,
    ],
)
def _deg_kernel(dst_hbm, zero_hbm, out_hbm, didx_v, ones_v, bounce_v, acc_s,
                sem):
    c = lax.axis_index("c")
    s = lax.axis_index("s")
    w = c * NS + s
    for k in range(CH // 16):
        ones_v[pl.ds(k * 16, 16)] = jnp.ones((16,), jnp.float32)
    pltpu.sync_copy(dst_hbm.at[pl.ds(pl.multiple_of(w * NCH, 8), NCH)],
                    didx_v)

    @pl.when(s == 0)
    def _():
        pltpu.sync_copy(zero_hbm, bounce_v)
        pltpu.sync_copy(bounce_v, acc_s.at[pl.ds(0, N)])
        pltpu.sync_copy(bounce_v.at[pl.ds(0, NPD - N)],
                        acc_s.at[pl.ds(N, NPD - N)])

    plsc.subcore_barrier()

    KG = 8

    def body(t, carry):
        for b in range(KG):
            pltpu.async_copy(ones_v, acc_s.at[didx_v.at[t * KG + b]], sem,
                             add=True)
        for b in range(KG):
            pltpu.make_async_copy(ones_v, acc_s.at[didx_v.at[t * KG + b]],
                                  sem).wait()
        return carry

    lax.fori_loop(0, NCH // KG, body, 0)
    plsc.subcore_barrier()

    @pl.when(s == 0)
    def _():
        pltpu.sync_copy(acc_s.at[pl.ds(0, N)], bounce_v)
        pltpu.sync_copy(bounce_v,
                        out_hbm.at[pl.ds(pl.multiple_of(c * N, 8), N)])


@functools.partial(
    pl.kernel,
    mesh=_mesh,
    out_type=jax.ShapeDtypeStruct((NC, N, D), jnp.float32),
    scratch_types=[
        pltpu.VMEM((CH,), jnp.int32),
        pltpu.VMEM((CH,), jnp.int32),
        pltpu.VMEM((CH, D), jnp.float32),
        pltpu.VMEM_SHARED((N, D), jnp.float32),
        pltpu.SemaphoreType.DMA,
    ],
)
def _agg_kernel(xs_hbm, src_hbm, dst_hbm, zero_hbm, out_hbm, si_v, di_v,
                rows_v, acc_s, sem):
    c = lax.axis_index("c")
    s = lax.axis_index("s")
    w = c * NS + s

    @pl.when(s == 0)
    def _():
        pltpu.sync_copy(zero_hbm, acc_s)

    plsc.subcore_barrier()
    base = w * (NCH * CH)

    def body(j, carry):
        off = base + j * CH
        pltpu.sync_copy(src_hbm.at[pl.ds(off, CH)], si_v)
        pltpu.async_copy(xs_hbm.at[si_v], rows_v, sem).wait()
        pltpu.sync_copy(dst_hbm.at[pl.ds(off, CH)], di_v)
        pltpu.sync_copy(rows_v, acc_s.at[di_v], add=True)
        return carry

    lax.fori_loop(0, NCH, body, 0)
    plsc.subcore_barrier()

    @pl.when(s == 0)
    def _():
        pltpu.sync_copy(acc_s, out_hbm.at[c])


def _scale_body(deg_ref, x_ref, xs_ref):
    dinv = lax.rsqrt(deg_ref[:, 0:1] + deg_ref[:, 1:2] + 1.0)
    xs_ref[0:N, :] = x_ref[...] * dinv
    xs_ref[N:NP, :] = jnp.zeros((NP - N, D), jnp.float32)


def _layer_body(acc_ref, deg_ref, xin_ref, w_ref, b_ref, h_ref, xs_ref):
    dinv = lax.rsqrt(deg_ref[:, 0:1] + deg_ref[:, 1:2] + 1.0)
    agg = dinv * (acc_ref[0] + acc_ref[1]) + (dinv * dinv) * xin_ref[...]
    h = jnp.dot(agg, w_ref[...], preferred_element_type=jnp.float32)
    h = jnp.maximum(h + b_ref[...], 0.0)
    h_ref[...] = h
    xs_ref[0:N, :] = h * dinv
    xs_ref[N:NP, :] = jnp.zeros((NP - N, D), jnp.float32)


def _final_body(acc_ref, deg_ref, h1_ref, w_ref, b_ref, batch_ref, lw_ref,
                lb_ref, out_ref):
    dinv = lax.rsqrt(deg_ref[:, 0:1] + deg_ref[:, 1:2] + 1.0)
    agg = dinv * (acc_ref[0] + acc_ref[1]) + (dinv * dinv) * h1_ref[...]
    h2 = jnp.dot(agg, w_ref[...], preferred_element_type=jnp.float32)
    h2 = jnp.maximum(h2 + b_ref[...], 0.0)
    gid = lax.broadcasted_iota(jnp.int32, (G, N), 0)
    onehot = (jnp.broadcast_to(batch_ref[...], (G, N)) == gid)
    onehot = onehot.astype(jnp.float32)
    sums = jnp.dot(onehot, h2, preferred_element_type=jnp.float32)
    counts = jnp.sum(onehot, axis=1, keepdims=True)
    pooled = sums / jnp.maximum(counts, 1.0)
    out_ref[...] = (
        jnp.dot(pooled, lw_ref[...], preferred_element_type=jnp.float32)
        + lb_ref[...])


_scale_call = pl.pallas_call(
    _scale_body, out_shape=jax.ShapeDtypeStruct((NP, D), jnp.float32))

_layer_call = pl.pallas_call(
    _layer_body,
    out_shape=(jax.ShapeDtypeStruct((N, D), jnp.float32),
               jax.ShapeDtypeStruct((NP, D), jnp.float32)))

_final_call = pl.pallas_call(
    _final_body, out_shape=jax.ShapeDtypeStruct((G, 1), jnp.float32))


@jax.jit
def kernel(x, edge_index, batch, W1, b1, W2, b2, lin_W, lin_b):
    src = edge_index[0]
    dst = edge_index[1]
    pad = E_PAD - E
    spread = jnp.arange(pad, dtype=jnp.int32)
    src1d = jnp.concatenate([src, jnp.full((pad,), N, jnp.int32)])
    dst2d_deg = jnp.concatenate([dst, N + spread % (NPD - N)]).reshape(
        NW * NCH, CH)
    dst1d = jnp.concatenate([dst, spread % N])
    zero1 = jnp.zeros((N,), jnp.float32)
    zero2 = jnp.zeros((N, D), jnp.float32)
    deg_t = _deg_kernel(dst2d_deg, zero1).reshape(NC, N).T
    xs1 = _scale_call(deg_t, x)
    acc1 = _agg_kernel(xs1, src1d, dst1d, zero2)
    h1, xs2 = _layer_call(acc1, deg_t, x, W1, b1)
    acc2 = _agg_kernel(xs2, src1d, dst1d, zero2)
    return _final_call(acc2, deg_t, h1, W2, b2, batch.reshape(1, N),
                       lin_W, lin_b)


# speedup vs baseline: 3.4874x; 3.4874x over previous
import functools

import jax
import jax.numpy as jnp
from jax import lax
from jax.experimental import pallas as pl
from jax.experimental.pallas import tpu as pltpu
from jax.experimental.pallas import tpu_sc as plsc

N = 10000
E = 320000
D = 128
G = 16
NC, NS = 2, 16
NW = NC * NS
CH = 128
NCH = 80
E_W = NCH * CH
E_PAD = NW * E_W
EW_RAW = E // NW
NFULL = EW_RAW // CH
ETAIL = EW_RAW - NFULL * CH
NP = N + 16
NPD = N + 640

_mesh = plsc.VectorSubcoreMesh(core_axis_name="c", subcore_axis_name="s")


@functools.partial(
    pl.kernel,
    mesh=_mesh,
    out_type=jax.ShapeDtypeStruct((NC * N,), jnp.float32),
    scratch_types=[
        pltpu.VMEM((NCH, CH), jnp.int32),
        pltpu.VMEM((CH,), jnp.float32),
        pltpu.VMEM((N,), jnp.float32),
        pltpu.VMEM_SHARED((NPD,), jnp.float32),
        pltpu.SemaphoreType.DMA,
    ],
)
def _deg_kernel(dst_hbm, zero_hbm, out_hbm, didx_v, ones_v, bounce_v, acc_s,
                sem):
    c = lax.axis_index("c")
    s = lax.axis_index("s")
    w = c * NS + s
    for k in range(CH // 16):
        ones_v[pl.ds(k * 16, 16)] = jnp.ones((16,), jnp.float32)
    pltpu.sync_copy(dst_hbm.at[pl.ds(pl.multiple_of(w * NCH, 8), NCH)],
                    didx_v)

    @pl.when(s == 0)
    def _():
        pltpu.sync_copy(zero_hbm, bounce_v)
        pltpu.sync_copy(bounce_v, acc_s.at[pl.ds(0, N)])
        pltpu.sync_copy(bounce_v.at[pl.ds(0, NPD - N)],
                        acc_s.at[pl.ds(N, NPD - N)])

    plsc.subcore_barrier()

    KG = 8

    def body(t, carry):
        for b in range(KG):
            pltpu.async_copy(ones_v, acc_s.at[didx_v.at[t * KG + b]], sem,
                             add=True)
        for b in range(KG):
            pltpu.make_async_copy(ones_v, acc_s.at[didx_v.at[t * KG + b]],
                                  sem).wait()
        return carry

    lax.fori_loop(0, NCH // KG, body, 0)
    plsc.subcore_barrier()

    @pl.when(s == 0)
    def _():
        pltpu.sync_copy(acc_s.at[pl.ds(0, N)], bounce_v)
        pltpu.sync_copy(bounce_v,
                        out_hbm.at[pl.ds(pl.multiple_of(c * N, 8), N)])


@functools.partial(
    pl.kernel,
    mesh=_mesh,
    out_type=jax.ShapeDtypeStruct((NC, N, D), jnp.float32),
    scratch_types=[
        pltpu.VMEM((CH,), jnp.int32),
        pltpu.VMEM((CH,), jnp.int32),
        pltpu.VMEM((CH,), jnp.int32),
        pltpu.VMEM((CH,), jnp.int32),
        pltpu.VMEM((ETAIL,), jnp.int32),
        pltpu.VMEM((ETAIL,), jnp.int32),
        pltpu.VMEM((CH, D), jnp.float32),
        pltpu.VMEM((CH, D), jnp.float32),
        pltpu.VMEM((ETAIL, D), jnp.float32),
        pltpu.VMEM_SHARED((N, D), jnp.float32),
        pltpu.SemaphoreType.DMA,
        pltpu.SemaphoreType.DMA,
    ],
)
def _agg_kernel(xs_hbm, src_hbm, dst_hbm, zero_hbm, out_hbm, si0_v, si1_v,
                di0_v, di1_v, sit_v, dit_v, rows0_v, rows1_v, rowst_v, acc_s,
                sem0, sem1):
    c = lax.axis_index("c")
    s = lax.axis_index("s")
    w = c * NS + s

    @pl.when(s == 0)
    def _():
        pltpu.sync_copy(zero_hbm, acc_s)

    plsc.subcore_barrier()
    base = w * EW_RAW
    NPAIR = NFULL // 2

    pltpu.sync_copy(src_hbm.at[pl.ds(base, CH)], si0_v)
    pltpu.async_copy(xs_hbm.at[si0_v], rows0_v, sem0)

    def body(i, carry):
        g0 = i * 2
        g1 = g0 + 1
        pltpu.sync_copy(src_hbm.at[pl.ds(base + g1 * CH, CH)], si1_v)
        pltpu.async_copy(xs_hbm.at[si1_v], rows1_v, sem1)
        pltpu.sync_copy(dst_hbm.at[pl.ds(base + g0 * CH, CH)], di0_v)
        pltpu.make_async_copy(xs_hbm.at[si0_v], rows0_v, sem0).wait()
        pltpu.sync_copy(rows0_v, acc_s.at[di0_v], add=True)

        @pl.when(i < NPAIR - 1)
        def _():
            pltpu.sync_copy(src_hbm.at[pl.ds(base + (g0 + 2) * CH, CH)],
                            si0_v)
            pltpu.async_copy(xs_hbm.at[si0_v], rows0_v, sem0)

        pltpu.sync_copy(dst_hbm.at[pl.ds(base + g1 * CH, CH)], di1_v)
        pltpu.make_async_copy(xs_hbm.at[si1_v], rows1_v, sem1).wait()
        pltpu.sync_copy(rows1_v, acc_s.at[di1_v], add=True)
        return carry

    lax.fori_loop(0, NPAIR, body, 0)
    off = base + NFULL * CH
    pltpu.sync_copy(src_hbm.at[pl.ds(off, ETAIL)], sit_v)
    pltpu.async_copy(xs_hbm.at[sit_v], rowst_v, sem0).wait()
    pltpu.sync_copy(dst_hbm.at[pl.ds(off, ETAIL)], dit_v)
    pltpu.sync_copy(rowst_v, acc_s.at[dit_v], add=True)
    plsc.subcore_barrier()

    @pl.when(s == 0)
    def _():
        pltpu.sync_copy(acc_s, out_hbm.at[c])


def _scale_body(deg_ref, x_ref, xs_ref):
    dinv = lax.rsqrt(deg_ref[:, 0:1] + deg_ref[:, 1:2] + 1.0)
    xs_ref[...] = x_ref[...] * dinv


def _layer_body(acc_ref, deg_ref, xin_ref, w_ref, b_ref, h_ref, xs_ref):
    dinv = lax.rsqrt(deg_ref[:, 0:1] + deg_ref[:, 1:2] + 1.0)
    agg = dinv * (acc_ref[0] + acc_ref[1]) + (dinv * dinv) * xin_ref[...]
    h = jnp.dot(agg, w_ref[...], preferred_element_type=jnp.float32)
    h = jnp.maximum(h + b_ref[...], 0.0)
    h_ref[...] = h
    xs_ref[...] = h * dinv


def _final_body(acc_ref, deg_ref, h1_ref, w_ref, b_ref, batch_ref, lw_ref,
                lb_ref, out_ref):
    dinv = lax.rsqrt(deg_ref[:, 0:1] + deg_ref[:, 1:2] + 1.0)
    agg = dinv * (acc_ref[0] + acc_ref[1]) + (dinv * dinv) * h1_ref[...]
    h2 = jnp.dot(agg, w_ref[...], preferred_element_type=jnp.float32)
    h2 = jnp.maximum(h2 + b_ref[...], 0.0)
    gid = lax.broadcasted_iota(jnp.int32, (G, N), 0)
    onehot = (jnp.broadcast_to(batch_ref[...], (G, N)) == gid)
    onehot = onehot.astype(jnp.float32)
    sums = jnp.dot(onehot, h2, preferred_element_type=jnp.float32)
    counts = jnp.sum(onehot, axis=1, keepdims=True)
    pooled = sums / jnp.maximum(counts, 1.0)
    out_ref[...] = (
        jnp.dot(pooled, lw_ref[...], preferred_element_type=jnp.float32)
        + lb_ref[...])


_scale_call = pl.pallas_call(
    _scale_body, out_shape=jax.ShapeDtypeStruct((N, D), jnp.float32))

_layer_call = pl.pallas_call(
    _layer_body,
    out_shape=(jax.ShapeDtypeStruct((N, D), jnp.float32),
               jax.ShapeDtypeStruct((N, D), jnp.float32)))

_final_call = pl.pallas_call(
    _final_body, out_shape=jax.ShapeDtypeStruct((G, 1), jnp.float32))


@jax.jit
def kernel(x, edge_index, batch, W1, b1, W2, b2, lin_W, lin_b):
    src = edge_index[0]
    dst = edge_index[1]
    pad = E_PAD - E
    spread = jnp.arange(pad, dtype=jnp.int32)
    dst2d_deg = jnp.concatenate([dst, N + spread % (NPD - N)]).reshape(
        NW * NCH, CH)
    src1d = src
    dst1d = dst
    zero1 = jnp.zeros((N,), jnp.float32)
    zero2 = jnp.zeros((N, D), jnp.float32)
    deg_t = _deg_kernel(dst2d_deg, zero1).reshape(NC, N).T
    xs1 = _scale_call(deg_t, x)
    acc1 = _agg_kernel(xs1, src1d, dst1d, zero2)
    h1, xs2 = _layer_call(acc1, deg_t, x, W1, b1)
    acc2 = _agg_kernel(xs2, src1d, dst1d, zero2)
    return _final_call(acc2, deg_t, h1, W2, b2, batch.reshape(1, N),
                       lin_W, lin_b)


# speedup vs baseline: 4.3953x; 1.2603x over previous
import functools

import jax
import jax.numpy as jnp
from jax import lax
from jax.experimental import pallas as pl
from jax.experimental.pallas import tpu as pltpu
from jax.experimental.pallas import tpu_sc as plsc

N = 10000
E = 320000
D = 128
G = 16
NC, NS = 2, 16
NW = NC * NS
CH = 128
NCH = 80
E_W = NCH * CH
E_PAD = NW * E_W
EW_RAW = E // NW
NFULL = EW_RAW // CH
ETAIL = EW_RAW - NFULL * CH
NP = N + 16
NPD = N + 640

_mesh = plsc.VectorSubcoreMesh(core_axis_name="c", subcore_axis_name="s")


@functools.partial(
    pl.kernel,
    mesh=_mesh,
    out_type=jax.ShapeDtypeStruct((NC * N,), jnp.float32),
    scratch_types=[
        pltpu.VMEM((NCH, CH), jnp.int32),
        pltpu.VMEM((CH,), jnp.float32),
        pltpu.VMEM((N,), jnp.float32),
        pltpu.VMEM_SHARED((NPD,), jnp.float32),
        pltpu.SemaphoreType.DMA,
    ],
)
def _deg_kernel(dst_hbm, zero_hbm, out_hbm, didx_v, ones_v, bounce_v, acc_s,
                sem):
    c = lax.axis_index("c")
    s = lax.axis_index("s")
    w = c * NS + s
    for k in range(CH // 16):
        ones_v[pl.ds(k * 16, 16)] = jnp.ones((16,), jnp.float32)
    pltpu.sync_copy(dst_hbm.at[pl.ds(pl.multiple_of(w * NCH, 8), NCH)],
                    didx_v)

    @pl.when(s == 0)
    def _():
        pltpu.sync_copy(zero_hbm, bounce_v)
        pltpu.sync_copy(bounce_v, acc_s.at[pl.ds(0, N)])
        pltpu.sync_copy(bounce_v.at[pl.ds(0, NPD - N)],
                        acc_s.at[pl.ds(N, NPD - N)])

    plsc.subcore_barrier()

    KG = 8

    def body(t, carry):
        for b in range(KG):
            pltpu.async_copy(ones_v, acc_s.at[didx_v.at[t * KG + b]], sem,
                             add=True)
        for b in range(KG):
            pltpu.make_async_copy(ones_v, acc_s.at[didx_v.at[t * KG + b]],
                                  sem).wait()
        return carry

    lax.fori_loop(0, NCH // KG, body, 0)
    plsc.subcore_barrier()

    @pl.when(s == 0)
    def _():
        pltpu.sync_copy(acc_s.at[pl.ds(0, N)], bounce_v)
        pltpu.sync_copy(bounce_v,
                        out_hbm.at[pl.ds(pl.multiple_of(c * N, 8), N)])


@functools.partial(
    pl.kernel,
    mesh=_mesh,
    out_type=jax.ShapeDtypeStruct((NC, N, D), jnp.float32),
    scratch_types=[
        pltpu.VMEM((CH,), jnp.int32),
        pltpu.VMEM((CH,), jnp.int32),
        pltpu.VMEM((CH,), jnp.int32),
        pltpu.VMEM((CH,), jnp.int32),
        pltpu.VMEM((ETAIL,), jnp.int32),
        pltpu.VMEM((ETAIL,), jnp.int32),
        pltpu.VMEM((CH, D), jnp.float32),
        pltpu.VMEM((CH, D), jnp.float32),
        pltpu.VMEM((ETAIL, D), jnp.float32),
        pltpu.VMEM_SHARED((N, D), jnp.float32),
        pltpu.SemaphoreType.DMA,
        pltpu.SemaphoreType.DMA,
        pltpu.SemaphoreType.DMA,
        pltpu.SemaphoreType.DMA,
        pltpu.SemaphoreType.DMA,
        pltpu.SemaphoreType.DMA,
    ],
)
def _agg_kernel(xs_hbm, src_hbm, dst_hbm, zero_hbm, out_hbm, si0_v, si1_v,
                di0_v, di1_v, sit_v, dit_v, rows0_v, rows1_v, rowst_v, acc_s,
                sem0, sem1, semd0, semd1, semsi0, semsi1):
    c = lax.axis_index("c")
    s = lax.axis_index("s")
    w = c * NS + s

    @pl.when(s == 0)
    def _():
        pltpu.sync_copy(zero_hbm, acc_s)

    plsc.subcore_barrier()
    base = w * EW_RAW
    NPAIR = NFULL // 2

    pltpu.sync_copy(src_hbm.at[pl.ds(base, CH)], si0_v)
    pltpu.async_copy(xs_hbm.at[si0_v], rows0_v, sem0)
    pltpu.async_copy(dst_hbm.at[pl.ds(base, CH)], di0_v, semd0)
    pltpu.async_copy(src_hbm.at[pl.ds(base + CH, CH)], si1_v, semsi1)

    def body(i, carry):
        g0 = i * 2
        g1 = g0 + 1
        pltpu.make_async_copy(src_hbm.at[pl.ds(base + g1 * CH, CH)], si1_v,
                              semsi1).wait()
        pltpu.async_copy(xs_hbm.at[si1_v], rows1_v, sem1)
        pltpu.async_copy(dst_hbm.at[pl.ds(base + g1 * CH, CH)], di1_v, semd1)
        pltpu.make_async_copy(xs_hbm.at[si0_v], rows0_v, sem0).wait()

        @pl.when(i < NPAIR - 1)
        def _():
            pltpu.async_copy(src_hbm.at[pl.ds(base + (g0 + 2) * CH, CH)],
                             si0_v, semsi0)

        pltpu.make_async_copy(dst_hbm.at[pl.ds(base + g0 * CH, CH)], di0_v,
                              semd0).wait()
        pltpu.sync_copy(rows0_v, acc_s.at[di0_v], add=True)

        @pl.when(i < NPAIR - 1)
        def _():
            pltpu.make_async_copy(src_hbm.at[pl.ds(base + (g0 + 2) * CH, CH)],
                                  si0_v, semsi0).wait()
            pltpu.async_copy(xs_hbm.at[si0_v], rows0_v, sem0)
            pltpu.async_copy(dst_hbm.at[pl.ds(base + (g0 + 2) * CH, CH)],
                             di0_v, semd0)

        pltpu.make_async_copy(dst_hbm.at[pl.ds(base + g1 * CH, CH)], di1_v,
                              semd1).wait()
        pltpu.make_async_copy(xs_hbm.at[si1_v], rows1_v, sem1).wait()

        @pl.when(i < NPAIR - 1)
        def _():
            pltpu.async_copy(src_hbm.at[pl.ds(base + (g1 + 2) * CH, CH)],
                             si1_v, semsi1)

        pltpu.sync_copy(rows1_v, acc_s.at[di1_v], add=True)
        return carry

    lax.fori_loop(0, NPAIR, body, 0)
    off = base + NFULL * CH
    pltpu.sync_copy(src_hbm.at[pl.ds(off, ETAIL)], sit_v)
    pltpu.async_copy(xs_hbm.at[sit_v], rowst_v, sem0).wait()
    pltpu.sync_copy(dst_hbm.at[pl.ds(off, ETAIL)], dit_v)
    pltpu.sync_copy(rowst_v, acc_s.at[dit_v], add=True)
    plsc.subcore_barrier()

    @pl.when(s == 0)
    def _():
        pltpu.sync_copy(acc_s, out_hbm.at[c])


def _scale_body(deg_ref, x_ref, xs_ref):
    dinv = lax.rsqrt(deg_ref[:, 0:1] + deg_ref[:, 1:2] + 1.0)
    xs_ref[...] = x_ref[...] * dinv


def _layer_body(acc_ref, deg_ref, xin_ref, w_ref, b_ref, h_ref, xs_ref):
    dinv = lax.rsqrt(deg_ref[:, 0:1] + deg_ref[:, 1:2] + 1.0)
    agg = dinv * (acc_ref[0] + acc_ref[1]) + (dinv * dinv) * xin_ref[...]
    h = jnp.dot(agg, w_ref[...], preferred_element_type=jnp.float32)
    h = jnp.maximum(h + b_ref[...], 0.0)
    h_ref[...] = h
    xs_ref[...] = h * dinv


def _final_body(acc_ref, deg_ref, h1_ref, w_ref, b_ref, batch_ref, lw_ref,
                lb_ref, out_ref):
    dinv = lax.rsqrt(deg_ref[:, 0:1] + deg_ref[:, 1:2] + 1.0)
    agg = dinv * (acc_ref[0] + acc_ref[1]) + (dinv * dinv) * h1_ref[...]
    h2 = jnp.dot(agg, w_ref[...], preferred_element_type=jnp.float32)
    h2 = jnp.maximum(h2 + b_ref[...], 0.0)
    gid = lax.broadcasted_iota(jnp.int32, (G, N), 0)
    onehot = (jnp.broadcast_to(batch_ref[...], (G, N)) == gid)
    onehot = onehot.astype(jnp.float32)
    sums = jnp.dot(onehot, h2, preferred_element_type=jnp.float32)
    counts = jnp.sum(onehot, axis=1, keepdims=True)
    pooled = sums / jnp.maximum(counts, 1.0)
    out_ref[...] = (
        jnp.dot(pooled, lw_ref[...], preferred_element_type=jnp.float32)
        + lb_ref[...])


_scale_call = pl.pallas_call(
    _scale_body, out_shape=jax.ShapeDtypeStruct((N, D), jnp.float32))

_layer_call = pl.pallas_call(
    _layer_body,
    out_shape=(jax.ShapeDtypeStruct((N, D), jnp.float32),
               jax.ShapeDtypeStruct((N, D), jnp.float32)))

_final_call = pl.pallas_call(
    _final_body, out_shape=jax.ShapeDtypeStruct((G, 1), jnp.float32))


@jax.jit
def kernel(x, edge_index, batch, W1, b1, W2, b2, lin_W, lin_b):
    src = edge_index[0]
    dst = edge_index[1]
    pad = E_PAD - E
    spread = jnp.arange(pad, dtype=jnp.int32)
    dst2d_deg = jnp.concatenate([dst, N + spread % (NPD - N)]).reshape(
        NW * NCH, CH)
    src1d = src
    dst1d = dst
    zero1 = jnp.zeros((N,), jnp.float32)
    zero2 = jnp.zeros((N, D), jnp.float32)
    deg_t = _deg_kernel(dst2d_deg, zero1).reshape(NC, N).T
    xs1 = _scale_call(deg_t, x)
    acc1 = _agg_kernel(xs1, src1d, dst1d, zero2)
    h1, xs2 = _layer_call(acc1, deg_t, x, W1, b1)
    acc2 = _agg_kernel(xs2, src1d, dst1d, zero2)
    return _final_call(acc2, deg_t, h1, W2, b2, batch.reshape(1, N),
                       lin_W, lin_b)


# speedup vs baseline: 4.4003x; 1.0011x over previous
import functools

import jax
import jax.numpy as jnp
from jax import lax
from jax.experimental import pallas as pl
from jax.experimental.pallas import tpu as pltpu
from jax.experimental.pallas import tpu_sc as plsc

N = 10000
E = 320000
D = 128
G = 16
NC, NS = 2, 16
NW = NC * NS
CH = 128
NCH = 80
E_W = NCH * CH
E_PAD = NW * E_W
EW_RAW = E // NW
NFULL = EW_RAW // CH
ETAIL = EW_RAW - NFULL * CH
NPD = N + 640

_mesh = plsc.VectorSubcoreMesh(core_axis_name="c", subcore_axis_name="s")


@functools.partial(
    pl.kernel,
    mesh=_mesh,
    out_type=jax.ShapeDtypeStruct((NC * N,), jnp.float32),
    scratch_types=[
        pltpu.VMEM((NCH, CH), jnp.int32),
        pltpu.VMEM((CH,), jnp.float32),
        pltpu.VMEM((N,), jnp.float32),
        pltpu.VMEM_SHARED((NPD,), jnp.float32),
        pltpu.SemaphoreType.DMA,
    ],
)
def _deg_kernel(dst_hbm, zero_hbm, out_hbm, didx_v, ones_v, bounce_v, acc_s,
                sem):
    c = lax.axis_index("c")
    s = lax.axis_index("s")
    w = c * NS + s
    for k in range(CH // 16):
        ones_v[pl.ds(k * 16, 16)] = jnp.ones((16,), jnp.float32)
    pltpu.sync_copy(dst_hbm.at[pl.ds(pl.multiple_of(w * NCH, 8), NCH)],
                    didx_v)

    @pl.when(s == 0)
    def _():
        pltpu.sync_copy(zero_hbm, bounce_v)
        pltpu.sync_copy(bounce_v, acc_s.at[pl.ds(0, N)])
        pltpu.sync_copy(bounce_v.at[pl.ds(0, NPD - N)],
                        acc_s.at[pl.ds(N, NPD - N)])

    plsc.subcore_barrier()

    KG = 8

    def body(t, carry):
        for b in range(KG):
            pltpu.async_copy(ones_v, acc_s.at[didx_v.at[t * KG + b]], sem,
                             add=True)
        for b in range(KG):
            pltpu.make_async_copy(ones_v, acc_s.at[didx_v.at[t * KG + b]],
                                  sem).wait()
        return carry

    lax.fori_loop(0, NCH // KG, body, 0)
    plsc.subcore_barrier()

    @pl.when(s == 0)
    def _():
        pltpu.sync_copy(acc_s.at[pl.ds(0, N)], bounce_v)
        pltpu.sync_copy(bounce_v,
                        out_hbm.at[pl.ds(pl.multiple_of(c * N, 8), N)])


@functools.partial(
    pl.kernel,
    mesh=_mesh,
    out_type=jax.ShapeDtypeStruct((NC, N, D), jnp.float32),
    scratch_types=[
        pltpu.VMEM((CH,), jnp.int32),
        pltpu.VMEM((CH,), jnp.int32),
        pltpu.VMEM((CH,), jnp.int32),
        pltpu.VMEM((CH,), jnp.int32),
        pltpu.VMEM((ETAIL,), jnp.int32),
        pltpu.VMEM((ETAIL,), jnp.int32),
        pltpu.VMEM((CH, D), jnp.float32),
        pltpu.VMEM((CH, D), jnp.float32),
        pltpu.VMEM((ETAIL, D), jnp.float32),
        pltpu.VMEM_SHARED((N, D), jnp.float32),
        pltpu.SemaphoreType.DMA,
        pltpu.SemaphoreType.DMA,
        pltpu.SemaphoreType.DMA,
        pltpu.SemaphoreType.DMA,
        pltpu.SemaphoreType.DMA,
        pltpu.SemaphoreType.DMA,
    ],
)
def _agg_kernel(xs_hbm, src_hbm, dst_hbm, zero_hbm, out_hbm, si0_v, si1_v,
                di0_v, di1_v, sit_v, dit_v, rows0_v, rows1_v, rowst_v, acc_s,
                sem0, sem1, semd0, semd1, semsi0, semsi1):
    c = lax.axis_index("c")
    s = lax.axis_index("s")
    w = c * NS + s

    @pl.when(s == 0)
    def _():
        pltpu.sync_copy(zero_hbm, acc_s)

    plsc.subcore_barrier()
    base = w * EW_RAW
    NPAIR = NFULL // 2

    pltpu.sync_copy(src_hbm.at[pl.ds(base, CH)], si0_v)
    pltpu.async_copy(xs_hbm.at[si0_v], rows0_v, sem0)
    pltpu.async_copy(dst_hbm.at[pl.ds(base, CH)], di0_v, semd0)
    pltpu.async_copy(src_hbm.at[pl.ds(base + CH, CH)], si1_v, semsi1)

    def body(i, carry):
        g0 = i * 2
        g1 = g0 + 1
        pltpu.make_async_copy(src_hbm.at[pl.ds(base + g1 * CH, CH)], si1_v,
                              semsi1).wait()
        pltpu.async_copy(xs_hbm.at[si1_v], rows1_v, sem1)
        pltpu.async_copy(dst_hbm.at[pl.ds(base + g1 * CH, CH)], di1_v, semd1)
        pltpu.make_async_copy(xs_hbm.at[si0_v], rows0_v, sem0).wait()

        @pl.when(i < NPAIR - 1)
        def _():
            pltpu.async_copy(src_hbm.at[pl.ds(base + (g0 + 2) * CH, CH)],
                             si0_v, semsi0)

        pltpu.make_async_copy(dst_hbm.at[pl.ds(base + g0 * CH, CH)], di0_v,
                              semd0).wait()
        pltpu.sync_copy(rows0_v, acc_s.at[di0_v], add=True)

        @pl.when(i < NPAIR - 1)
        def _():
            pltpu.make_async_copy(src_hbm.at[pl.ds(base + (g0 + 2) * CH, CH)],
                                  si0_v, semsi0).wait()
            pltpu.async_copy(xs_hbm.at[si0_v], rows0_v, sem0)
            pltpu.async_copy(dst_hbm.at[pl.ds(base + (g0 + 2) * CH, CH)],
                             di0_v, semd0)

        pltpu.make_async_copy(dst_hbm.at[pl.ds(base + g1 * CH, CH)], di1_v,
                              semd1).wait()
        pltpu.make_async_copy(xs_hbm.at[si1_v], rows1_v, sem1).wait()

        @pl.when(i < NPAIR - 1)
        def _():
            pltpu.async_copy(src_hbm.at[pl.ds(base + (g1 + 2) * CH, CH)],
                             si1_v, semsi1)

        pltpu.sync_copy(rows1_v, acc_s.at[di1_v], add=True)
        return carry

    lax.fori_loop(0, NPAIR, body, 0)
    off = base + NFULL * CH
    pltpu.sync_copy(src_hbm.at[pl.ds(off, ETAIL)], sit_v)
    pltpu.async_copy(xs_hbm.at[sit_v], rowst_v, sem0).wait()
    pltpu.sync_copy(dst_hbm.at[pl.ds(off, ETAIL)], dit_v)
    pltpu.sync_copy(rowst_v, acc_s.at[dit_v], add=True)
    plsc.subcore_barrier()

    @pl.when(s == 0)
    def _():
        pltpu.sync_copy(acc_s, out_hbm.at[c])


def _scale_body(deg_ref, x_ref, xs_ref):
    dinv = lax.rsqrt(deg_ref[:, 0:1] + deg_ref[:, 1:2] + 1.0)
    xs_ref[...] = x_ref[...] * dinv


def _layer_body(acc_ref, deg_ref, xin_ref, w_ref, b_ref, h_ref, xs_ref):
    dinv = lax.rsqrt(deg_ref[:, 0:1] + deg_ref[:, 1:2] + 1.0)
    agg = dinv * (acc_ref[0] + acc_ref[1]) + (dinv * dinv) * xin_ref[...]
    h = jnp.dot(agg, w_ref[...], preferred_element_type=jnp.float32)
    h = jnp.maximum(h + b_ref[...], 0.0)
    h_ref[...] = h
    xs_ref[...] = h * dinv


def _final_body(acc_ref, deg_ref, h1_ref, w_ref, b_ref, batch_ref, lw_ref,
                lb_ref, out_ref):
    dinv = lax.rsqrt(deg_ref[:, 0:1] + deg_ref[:, 1:2] + 1.0)
    agg = dinv * (acc_ref[0] + acc_ref[1]) + (dinv * dinv) * h1_ref[...]
    h2 = jnp.dot(agg, w_ref[...], preferred_element_type=jnp.float32)
    h2 = jnp.maximum(h2 + b_ref[...], 0.0)
    gid = lax.broadcasted_iota(jnp.int32, (G, N), 0)
    onehot = (jnp.broadcast_to(batch_ref[...], (G, N)) == gid)
    onehot = onehot.astype(jnp.float32)
    sums = jnp.dot(onehot, h2, preferred_element_type=jnp.float32)
    counts = jnp.sum(onehot, axis=1, keepdims=True)
    pooled = sums / jnp.maximum(counts, 1.0)
    out_ref[...] = (
        jnp.dot(pooled, lw_ref[...], preferred_element_type=jnp.float32)
        + lb_ref[...])


_scale_call = pl.pallas_call(
    _scale_body, out_shape=jax.ShapeDtypeStruct((N, D), jnp.float32))

_layer_call = pl.pallas_call(
    _layer_body,
    out_shape=(jax.ShapeDtypeStruct((N, D), jnp.float32),
               jax.ShapeDtypeStruct((N, D), jnp.float32)))

_final_call = pl.pallas_call(
    _final_body, out_shape=jax.ShapeDtypeStruct((G, 1), jnp.float32))


@jax.jit
def kernel(x, edge_index, batch, W1, b1, W2, b2, lin_W, lin_b):
    src = edge_index[0]
    dst = edge_index[1]
    pad = E_PAD - E
    spread = jnp.arange(pad, dtype=jnp.int32)
    dst2d_deg = jnp.concatenate([dst, N + spread % (NPD - N)]).reshape(
        NW * NCH, CH)
    src1d = src
    dst1d = dst
    zero1 = jnp.zeros((N,), jnp.float32)
    zero2 = jnp.zeros((N, D), jnp.float32)
    deg_t = _deg_kernel(dst2d_deg, zero1).reshape(NC, N).T
    xs1 = _scale_call(deg_t, x)
    acc1 = _agg_kernel(xs1, src1d, dst1d, zero2)
    h1, xs2 = _layer_call(acc1, deg_t, x, W1, b1)
    acc2 = _agg_kernel(xs2, src1d, dst1d, zero2)
    return _final_call(acc2, deg_t, h1, W2, b2, batch.reshape(1, N),
                       lin_W, lin_b)
